# baseline (speedup 1.0000x reference)
import functools

import jax
import jax.numpy as jnp
from jax import lax
from jax.experimental import pallas as pl
from jax.experimental.pallas import tpu as pltpu
from jax.experimental.pallas import tpu_sc as plsc

B = 4
N = 2500
NP = 2560
BN = B * NP
H = 256
HEADS = 4
E = 40000
KT = 512
MT = 1280
NMT = NP // MT
NKT = NP // KT

ROWS_PER_TILE = 40
NTILES = 32
HALF = ROWS_PER_TILE * NTILES
ECHUNK = 4000
BUFW = ROWS_PER_TILE * NP


def _sc_build_c_body(src_hbm, tgt_hbm, c_hbm, buf, sstage0, sstage1,
                     tstage0, tstage1, dsem):
    sstage = (sstage0, sstage1)
    tstage = (tstage0, tstage1)
    cid = lax.axis_index("c")
    sid = lax.axis_index("s")
    wid = sid * 2 + cid
    ones = jnp.full((16,), 1.0, jnp.float32)
    zeros = jnp.zeros((16,), jnp.float32)
    lane = lax.iota(jnp.int32, 16)
    nchunks = E // ECHUNK

    def start_chunk(b, ch, par):
        eoff = b * E + ch * ECHUNK
        cs = pltpu.make_async_copy(
            src_hbm.at[pl.ds(eoff, ECHUNK)], sstage[par], dsem)
        cs.start()
        ct = pltpu.make_async_copy(
            tgt_hbm.at[pl.ds(eoff, ECHUNK)], tstage[par], dsem)
        ct.start()
        return cs, ct

    for p in range(2 * B):
        b = p // 2
        half = p % 2
        lo = half * HALF + wid * ROWS_PER_TILE

        pend = start_chunk(b, 0, 0)

        def zbody(i, _):
            buf[pl.ds(i * 64, 16)] = zeros
            buf[pl.ds(i * 64 + 16, 16)] = zeros
            buf[pl.ds(i * 64 + 32, 16)] = zeros
            buf[pl.ds(i * 64 + 48, 16)] = zeros
            return 0
        lax.fori_loop(0, BUFW // 64, zbody, 0)

        for g in range(3):
            j = g * 16 + lane
            didx = j * NP + lo + j
            dmask = j < ROWS_PER_TILE
            didx = jnp.where(dmask, didx, 0)
            plsc.addupdate_scatter(buf, [didx], ones, mask=dmask)

        for ch in range(nchunks):
            par = ch % 2
            pend[0].wait()
            pend[1].wait()
            if ch + 1 < nchunks:
                pend = start_chunk(b, ch + 1, 1 - par)

            def ebody(i, _):
                for u in range(2):
                    o = i * 32 + u * 16
                    tg = tstage[par][pl.ds(o, 16)]
                    sr = sstage[par][pl.ds(o, 16)]
                    ridx = tg - lo
                    m = ridx.astype(jnp.uint32) < ROWS_PER_TILE
                    fidx = jnp.where(m, ridx * NP + sr, 0)
                    plsc.addupdate_scatter(buf, [fidx], ones, mask=m)
                return 0
            lax.fori_loop(0, ECHUNK // 32, ebody, 0)

        row0 = b * NP + half * HALF + wid * ROWS_PER_TILE
        pltpu.sync_copy(buf, c_hbm.at[pl.ds(row0 * NP, BUFW)])


def _build_c(src_flat, tgt_flat):
    mesh = plsc.VectorSubcoreMesh(core_axis_name="c", subcore_axis_name="s")
    kern = functools.partial(
        pl.kernel,
        mesh=mesh,
        out_type=jax.ShapeDtypeStruct((BN * NP,), jnp.float32),
        scratch_types=[
            pltpu.VMEM((BUFW,), jnp.float32),
            pltpu.VMEM((ECHUNK,), jnp.int32),
            pltpu.VMEM((ECHUNK,), jnp.int32),
            pltpu.VMEM((ECHUNK,), jnp.int32),
            pltpu.VMEM((ECHUNK,), jnp.int32),
            pltpu.SemaphoreType.DMA,
        ],
        compiler_params=pltpu.CompilerParams(needs_layout_passes=False),
    )(_sc_build_c_body)
    return kern(src_flat, tgt_flat).reshape(BN, NP)


def _proj_body(x_ref, w_ref, asd_s_ref, asd_d_ref,
               xw_ref, east_ref, ebst_ref, alst_ref,
               ead_ref, ebd_ref, nad_ref, xs_ref, *, pad_input):
    if pad_input:
        xs_ref[pl.ds(0, N), :] = x_ref[0]
        xs_ref[pl.ds(N, NP - N), :] = jnp.zeros((NP - N, H), jnp.float32)
        x = xs_ref[...]
    else:
        x = x_ref[...]
    xw = jnp.dot(x, w_ref[...], preferred_element_type=jnp.float32)
    xw_ref[...] = xw
    als = jnp.dot(xw, asd_s_ref[...], preferred_element_type=jnp.float32)
    ald = jnp.dot(xw, asd_d_ref[...], preferred_element_type=jnp.float32)
    ms = jnp.max(als, axis=0, keepdims=True)
    t = ms + ald
    mp = jnp.where(t > 0, t, 0.2 * t)
    ead_ref[...] = jnp.exp(ald - mp).astype(jnp.bfloat16)
    ebd_ref[...] = jnp.exp(0.2 * ald - mp).astype(jnp.bfloat16)
    nad_ref[...] = (-ald).astype(jnp.bfloat16)
    alt = jnp.transpose(als)
    east_ref[...] = jnp.exp(alt[0:8, :]).astype(jnp.bfloat16)
    ebst_ref[...] = jnp.exp(0.2 * alt[0:8, :]).astype(jnp.bfloat16)
    alst_ref[...] = alt[0:8, :].astype(jnp.bfloat16)


def _run_proj(x, w, asd_s, asd_d, pad_input):
    f32 = jnp.float32
    bf16 = jnp.bfloat16
    out_shapes = (
        jax.ShapeDtypeStruct((BN, HEADS * H), f32),
        jax.ShapeDtypeStruct((B * 8, NP), bf16),
        jax.ShapeDtypeStruct((B * 8, NP), bf16),
        jax.ShapeDtypeStruct((B * 8, NP), bf16),
        jax.ShapeDtypeStruct((BN, 128), bf16),
        jax.ShapeDtypeStruct((BN, 128), bf16),
        jax.ShapeDtypeStruct((BN, 128), bf16),
    )
    grid = (B,)
    if pad_input:
        xspec = pl.BlockSpec((1, N, H), lambda b: (b, 0, 0))
    else:
        xspec = pl.BlockSpec((NP, H), lambda b: (b, 0))
    in_specs = [
        xspec,
        pl.BlockSpec((H, HEADS * H), lambda b: (0, 0)),
        pl.BlockSpec((HEADS * H, 128), lambda b: (0, 0)),
        pl.BlockSpec((HEADS * H, 128), lambda b: (0, 0)),
    ]
    out_specs = (
        pl.BlockSpec((NP, HEADS * H), lambda b: (b, 0)),
        pl.BlockSpec((8, NP), lambda b: (b, 0)),
        pl.BlockSpec((8, NP), lambda b: (b, 0)),
        pl.BlockSpec((8, NP), lambda b: (b, 0)),
        pl.BlockSpec((NP, 128), lambda b: (b, 0)),
        pl.BlockSpec((NP, 128), lambda b: (b, 0)),
        pl.BlockSpec((NP, 128), lambda b: (b, 0)),
    )
    return pl.pallas_call(
        functools.partial(_proj_body, pad_input=pad_input),
        grid=grid,
        in_specs=in_specs,
        out_specs=out_specs,
        out_shape=out_shapes,
        scratch_shapes=[pltpu.VMEM((NP, H), f32)],
        compiler_params=pltpu.CompilerParams(
            dimension_semantics=("arbitrary",)),
    )(x, w, asd_s, asd_d)


def _gat_body(c_ref, xw_ref, east_ref, ebst_ref, alst_ref,
              ead_ref, ebd_ref, nad_ref, bias_ref, lng_ref, lnb_ref,
              out_ref, acca_ref, accb_ref, ssuma_ref, ssumb_ref,
              *, apply_ln):
    k = pl.program_id(2)

    @pl.when(k == 0)
    def _init():
        acca_ref[...] = jnp.zeros_like(acca_ref)
        accb_ref[...] = jnp.zeros_like(accb_ref)
        ssuma_ref[...] = jnp.zeros_like(ssuma_ref)
        ssumb_ref[...] = jnp.zeros_like(ssumb_ref)

    c = c_ref[...].astype(jnp.bfloat16)
    xw = xw_ref[...]
    east = east_ref[...]
    ebst = ebst_ref[...]
    alst = alst_ref[...]
    nad = nad_ref[...]

    zero = jnp.zeros((), jnp.bfloat16)
    for h in range(HEADS):
        eas = east[h:h + 1, :]
        ebs = ebst[h:h + 1, :]
        als = alst[h:h + 1, :]
        nadh = nad[:, h:h + 1]
        cond = als > nadh
        ca = jnp.where(cond, c * eas, zero)
        cb = jnp.where(cond, zero, c * ebs)
        xwb = xw[:, h * H:(h + 1) * H].astype(jnp.bfloat16)
        acca_ref[:, h * H:(h + 1) * H] += jnp.dot(
            ca, xwb, preferred_element_type=jnp.float32)
        accb_ref[:, h * H:(h + 1) * H] += jnp.dot(
            cb, xwb, preferred_element_type=jnp.float32)
        pa = (ca[:, 0:128] + ca[:, 128:256]) + (ca[:, 256:384] + ca[:, 384:512])
        pb = (cb[:, 0:128] + cb[:, 128:256]) + (cb[:, 256:384] + cb[:, 384:512])
        ssuma_ref[:, h * 128:(h + 1) * 128] += pa.astype(jnp.float32)
        ssumb_ref[:, h * 128:(h + 1) * 128] += pb.astype(jnp.float32)

    @pl.when(k == NKT - 1)
    def _finalize():
        ead = ead_ref[...].astype(jnp.float32)
        ebd = ebd_ref[...].astype(jnp.float32)
        o = jnp.zeros((MT, H), jnp.float32)
        for h in range(HEADS):
            eadh = ead[:, h:h + 1]
            ebdh = ebd[:, h:h + 1]
            rs = (eadh * jnp.sum(ssuma_ref[:, h * 128:(h + 1) * 128], axis=1,
                                 keepdims=True)
                  + ebdh * jnp.sum(ssumb_ref[:, h * 128:(h + 1) * 128], axis=1,
                                   keepdims=True))
            oh = (eadh * acca_ref[:, h * H:(h + 1) * H]
                  + ebdh * accb_ref[:, h * H:(h + 1) * H])
            o = o + oh / (rs + 1e-16)
        o = o * (1.0 / HEADS) + bias_ref[0:1, :]
        if apply_ln:
            mu = jnp.mean(o, axis=1, keepdims=True)
            d = o - mu
            var = jnp.mean(d * d, axis=1, keepdims=True)
            o = d * jax.lax.rsqrt(var + 1e-5) * lng_ref[0:1, :] + lnb_ref[0:1, :]
            o = jnp.maximum(o, 0.0)
        out_ref[...] = o


def _run_gat(c, xw, east, ebst, alst, ead, ebd, nad, bias, lng, lnb,
             apply_ln):
    grid = (B, NMT, NKT)
    in_specs = [
        pl.BlockSpec((MT, KT), lambda b, m, k: (b * NMT + m, k)),
        pl.BlockSpec((KT, HEADS * H), lambda b, m, k: (b * NKT + k, 0)),
        pl.BlockSpec((8, KT), lambda b, m, k: (b, k)),
        pl.BlockSpec((8, KT), lambda b, m, k: (b, k)),
        pl.BlockSpec((8, KT), lambda b, m, k: (b, k)),
        pl.BlockSpec((MT, 128), lambda b, m, k: (b * NMT + m, 0)),
        pl.BlockSpec((MT, 128), lambda b, m, k: (b * NMT + m, 0)),
        pl.BlockSpec((MT, 128), lambda b, m, k: (b * NMT + m, 0)),
        pl.BlockSpec((8, H), lambda b, m, k: (0, 0)),
        pl.BlockSpec((8, H), lambda b, m, k: (0, 0)),
        pl.BlockSpec((8, H), lambda b, m, k: (0, 0)),
    ]
    out_specs = pl.BlockSpec((MT, H), lambda b, m, k: (b * NMT + m, 0))
    out_shape = jax.ShapeDtypeStruct((BN, H), jnp.float32)
    return pl.pallas_call(
        functools.partial(_gat_body, apply_ln=apply_ln),
        grid=grid,
        in_specs=in_specs,
        out_specs=out_specs,
        out_shape=out_shape,
        scratch_shapes=[
            pltpu.VMEM((MT, HEADS * H), jnp.float32),
            pltpu.VMEM((MT, HEADS * H), jnp.float32),
            pltpu.VMEM((MT, HEADS * 128), jnp.float32),
            pltpu.VMEM((MT, HEADS * 128), jnp.float32),
        ],
        compiler_params=pltpu.CompilerParams(
            dimension_semantics=("arbitrary", "arbitrary", "arbitrary")),
    )(c, xw, east, ebst, alst, ead, ebd, nad, bias, lng, lnb)


def _pack_att(att):
    m = jnp.zeros((HEADS * H, 128), jnp.float32)
    for h in range(HEADS):
        m = m.at[h * H:(h + 1) * H, h].set(att[h])
    return m


def _layer(x, c, w, att_src, att_dst, bias, lng, lnb, apply_ln,
           pad_input=False):
    asd_s = _pack_att(att_src)
    asd_d = _pack_att(att_dst)
    xw, east, ebst, alst, ead, ebd, nad = _run_proj(x, w, asd_s, asd_d,
                                                    pad_input)
    b2 = jnp.broadcast_to(bias[None, :], (8, H))
    g2 = jnp.broadcast_to(lng[None, :], (8, H))
    l2 = jnp.broadcast_to(lnb[None, :], (8, H))
    return _run_gat(c, xw, east, ebst, alst, ead, ebd, nad, b2, g2, l2,
                    apply_ln)


def kernel(node_embeddings, src_index, tgt_index,
           W1, att_src1, att_dst1, bias1,
           W2, att_src2, att_dst2, bias2,
           W3, att_src3, att_dst3, bias3,
           ln1_g, ln1_b, ln2_g, ln2_b):
    src_flat = src_index.reshape(-1).astype(jnp.int32)
    tgt_flat = tgt_index.reshape(-1).astype(jnp.int32)
    c = _build_c(src_flat, tgt_flat)
    x = _layer(node_embeddings, c, W1, att_src1, att_dst1, bias1,
               ln1_g, ln1_b, True, pad_input=True)
    x = _layer(x, c, W2, att_src2, att_dst2, bias2, ln2_g, ln2_b, True)
    x = _layer(x, c, W3, att_src3, att_dst3, bias3, ln1_g, ln1_b, False)
    return x.reshape(B, NP, H)[:, :N, :]


# speedup vs baseline: 1.1894x; 1.1894x over previous
import functools

import jax
import jax.numpy as jnp
from jax import lax
from jax.experimental import pallas as pl
from jax.experimental.pallas import tpu as pltpu
from jax.experimental.pallas import tpu_sc as plsc

B = 4
N = 2500
NP = 2560
BN = B * NP
H = 256
HEADS = 4
E = 40000
KT = 512
MT = 1280
NMT = NP // MT
NKT = NP // KT

ROWS_PER_TILE = 40
NTILES = 32
HALF = ROWS_PER_TILE * NTILES
ECHUNK = 4000
BUFW = ROWS_PER_TILE * NP


def _sc_build_c_body(src_hbm, tgt_hbm, c_hbm, buf, sstage0, sstage1,
                     tstage0, tstage1, dsem):
    sstage = (sstage0, sstage1)
    tstage = (tstage0, tstage1)
    cid = lax.axis_index("c")
    sid = lax.axis_index("s")
    wid = sid * 2 + cid
    ones = jnp.full((16,), 1.0, jnp.float32)
    zeros = jnp.zeros((16,), jnp.float32)
    lane = lax.iota(jnp.int32, 16)
    nchunks = E // ECHUNK

    def start_chunk(b, ch, par):
        eoff = b * E + ch * ECHUNK
        cs = pltpu.make_async_copy(
            src_hbm.at[pl.ds(eoff, ECHUNK)], sstage[par], dsem)
        cs.start()
        ct = pltpu.make_async_copy(
            tgt_hbm.at[pl.ds(eoff, ECHUNK)], tstage[par], dsem)
        ct.start()
        return cs, ct

    for p in range(2 * B):
        b = p // 2
        half = p % 2
        lo = half * HALF + wid * ROWS_PER_TILE

        pend = start_chunk(b, 0, 0)

        def zbody(i, _):
            buf[pl.ds(i * 64, 16)] = zeros
            buf[pl.ds(i * 64 + 16, 16)] = zeros
            buf[pl.ds(i * 64 + 32, 16)] = zeros
            buf[pl.ds(i * 64 + 48, 16)] = zeros
            return 0
        lax.fori_loop(0, BUFW // 64, zbody, 0)

        for g in range(3):
            j = g * 16 + lane
            didx = j * NP + lo + j
            dmask = j < ROWS_PER_TILE
            didx = jnp.where(dmask, didx, 0)
            plsc.addupdate_scatter(buf, [didx], ones, mask=dmask)

        for ch in range(nchunks):
            par = ch % 2
            pend[0].wait()
            pend[1].wait()
            if ch + 1 < nchunks:
                pend = start_chunk(b, ch + 1, 1 - par)

            def ebody(i, _):
                for u in range(2):
                    o = i * 32 + u * 16
                    tg = tstage[par][pl.ds(o, 16)]
                    sr = sstage[par][pl.ds(o, 16)]
                    ridx = tg - lo
                    m = ridx.astype(jnp.uint32) < ROWS_PER_TILE
                    fidx = jnp.where(m, ridx * NP + sr, 0)
                    plsc.addupdate_scatter(buf, [fidx], ones, mask=m)
                return 0
            lax.fori_loop(0, ECHUNK // 32, ebody, 0)

        row0 = b * NP + half * HALF + wid * ROWS_PER_TILE
        pltpu.sync_copy(buf, c_hbm.at[pl.ds(row0 * NP, BUFW)])


def _build_c(src_flat, tgt_flat):
    mesh = plsc.VectorSubcoreMesh(core_axis_name="c", subcore_axis_name="s")
    kern = functools.partial(
        pl.kernel,
        mesh=mesh,
        out_type=jax.ShapeDtypeStruct((BN * NP,), jnp.float32),
        scratch_types=[
            pltpu.VMEM((BUFW,), jnp.float32),
            pltpu.VMEM((ECHUNK,), jnp.int32),
            pltpu.VMEM((ECHUNK,), jnp.int32),
            pltpu.VMEM((ECHUNK,), jnp.int32),
            pltpu.VMEM((ECHUNK,), jnp.int32),
            pltpu.SemaphoreType.DMA,
        ],
        compiler_params=pltpu.CompilerParams(needs_layout_passes=False),
    )(_sc_build_c_body)
    return kern(src_flat, tgt_flat).reshape(BN, NP)


def _proj_body(x_ref, w_ref, asd_s_ref, asd_d_ref,
               xw_ref, east_ref, ebst_ref, alst_ref,
               ead_ref, ebd_ref, nad_ref, xs_ref, *, pad_input):
    if pad_input:
        xs_ref[pl.ds(0, N), :] = x_ref[0]
        xs_ref[pl.ds(N, NP - N), :] = jnp.zeros((NP - N, H), jnp.float32)
        x = xs_ref[...]
    else:
        x = x_ref[...]
    xw = jnp.dot(x, w_ref[...], preferred_element_type=jnp.float32)
    xw_ref[...] = xw
    als = jnp.dot(xw, asd_s_ref[...], preferred_element_type=jnp.float32)
    ald = jnp.dot(xw, asd_d_ref[...], preferred_element_type=jnp.float32)
    ms = jnp.max(als, axis=0, keepdims=True)
    t = ms + ald
    mp = jnp.where(t > 0, t, 0.2 * t)
    ead_ref[...] = jnp.exp(ald - mp).astype(jnp.bfloat16)
    ebd_ref[...] = jnp.exp(0.2 * ald - mp).astype(jnp.bfloat16)
    nad_ref[...] = (-ald).astype(jnp.bfloat16)
    alt = jnp.transpose(als)
    east_ref[...] = jnp.exp(alt[0:8, :]).astype(jnp.bfloat16)
    ebst_ref[...] = jnp.exp(0.2 * alt[0:8, :]).astype(jnp.bfloat16)
    alst_ref[...] = alt[0:8, :].astype(jnp.bfloat16)


def _run_proj(x, w, asd_s, asd_d, pad_input):
    f32 = jnp.float32
    bf16 = jnp.bfloat16
    out_shapes = (
        jax.ShapeDtypeStruct((BN, HEADS * H), f32),
        jax.ShapeDtypeStruct((B * 8, NP), bf16),
        jax.ShapeDtypeStruct((B * 8, NP), bf16),
        jax.ShapeDtypeStruct((B * 8, NP), bf16),
        jax.ShapeDtypeStruct((BN, 128), bf16),
        jax.ShapeDtypeStruct((BN, 128), bf16),
        jax.ShapeDtypeStruct((BN, 128), bf16),
    )
    grid = (B,)
    if pad_input:
        xspec = pl.BlockSpec((1, N, H), lambda b: (b, 0, 0))
    else:
        xspec = pl.BlockSpec((NP, H), lambda b: (b, 0))
    in_specs = [
        xspec,
        pl.BlockSpec((H, HEADS * H), lambda b: (0, 0)),
        pl.BlockSpec((HEADS * H, 128), lambda b: (0, 0)),
        pl.BlockSpec((HEADS * H, 128), lambda b: (0, 0)),
    ]
    out_specs = (
        pl.BlockSpec((NP, HEADS * H), lambda b: (b, 0)),
        pl.BlockSpec((8, NP), lambda b: (b, 0)),
        pl.BlockSpec((8, NP), lambda b: (b, 0)),
        pl.BlockSpec((8, NP), lambda b: (b, 0)),
        pl.BlockSpec((NP, 128), lambda b: (b, 0)),
        pl.BlockSpec((NP, 128), lambda b: (b, 0)),
        pl.BlockSpec((NP, 128), lambda b: (b, 0)),
    )
    return pl.pallas_call(
        functools.partial(_proj_body, pad_input=pad_input),
        grid=grid,
        in_specs=in_specs,
        out_specs=out_specs,
        out_shape=out_shapes,
        scratch_shapes=[pltpu.VMEM((NP, H), f32)],
        compiler_params=pltpu.CompilerParams(
            dimension_semantics=("arbitrary",)),
    )(x, w, asd_s, asd_d)


def _gat_body(c_ref, xw_ref, east_ref, ebst_ref, alst_ref,
              ead_ref, ebd_ref, nad_ref, bias_ref, lng_ref, lnb_ref,
              out_ref, acc_ref, ssum_ref, *, apply_ln, final):
    k = pl.program_id(1)

    @pl.when(k == 0)
    def _init():
        acc_ref[...] = jnp.zeros_like(acc_ref)
        ssum_ref[...] = jnp.zeros_like(ssum_ref)

    c = c_ref[...].astype(jnp.bfloat16)
    xw = xw_ref[...]
    east = east_ref[...]
    ebst = ebst_ref[...]
    alst = alst_ref[...]
    ead = ead_ref[...]
    ebd = ebd_ref[...]
    nad = nad_ref[...]

    for h in range(HEADS):
        eas = east[h:h + 1, :]
        ebs = ebst[h:h + 1, :]
        als = alst[h:h + 1, :]
        eadh = ead[:, h:h + 1]
        ebdh = ebd[:, h:h + 1]
        nadh = nad[:, h:h + 1]
        cond = als > nadh
        q = jnp.where(cond, eas * eadh, ebs * ebdh)
        a = c * q
        xwb = xw[:, h * H:(h + 1) * H].astype(jnp.bfloat16)
        acc_ref[:, h * H:(h + 1) * H] += jnp.dot(
            a, xwb, preferred_element_type=jnp.float32)
        part = (a[:, 0:128] + a[:, 128:256]) + (a[:, 256:384] + a[:, 384:512])
        ssum_ref[:, h * 128:(h + 1) * 128] += part.astype(jnp.float32)

    @pl.when(k == NKT - 1)
    def _finalize():
        o = jnp.zeros((NP, H), jnp.float32)
        for h in range(HEADS):
            rs = jnp.sum(ssum_ref[:, h * 128:(h + 1) * 128], axis=1,
                         keepdims=True)
            o = o + acc_ref[:, h * H:(h + 1) * H] / (rs + 1e-16)
        o = o * (1.0 / HEADS) + bias_ref[0:1, :]
        if apply_ln:
            mu = jnp.mean(o, axis=1, keepdims=True)
            d = o - mu
            var = jnp.mean(d * d, axis=1, keepdims=True)
            o = d * jax.lax.rsqrt(var + 1e-5) * lng_ref[0:1, :] + lnb_ref[0:1, :]
            o = jnp.maximum(o, 0.0)
        if final:
            out_ref[0] = o[0:N, :]
        else:
            out_ref[...] = o


def _run_gat(c, xw, east, ebst, alst, ead, ebd, nad, bias, lng, lnb,
             apply_ln, final):
    grid = (B, NKT)
    in_specs = [
        pl.BlockSpec((NP, KT), lambda b, k: (b, k)),
        pl.BlockSpec((KT, HEADS * H), lambda b, k: (b * NKT + k, 0)),
        pl.BlockSpec((8, KT), lambda b, k: (b, k)),
        pl.BlockSpec((8, KT), lambda b, k: (b, k)),
        pl.BlockSpec((8, KT), lambda b, k: (b, k)),
        pl.BlockSpec((NP, 128), lambda b, k: (b, 0)),
        pl.BlockSpec((NP, 128), lambda b, k: (b, 0)),
        pl.BlockSpec((NP, 128), lambda b, k: (b, 0)),
        pl.BlockSpec((8, H), lambda b, k: (0, 0)),
        pl.BlockSpec((8, H), lambda b, k: (0, 0)),
        pl.BlockSpec((8, H), lambda b, k: (0, 0)),
    ]
    if final:
        out_specs = pl.BlockSpec((1, N, H), lambda b, k: (b, 0, 0))
        out_shape = jax.ShapeDtypeStruct((B, N, H), jnp.float32)
    else:
        out_specs = pl.BlockSpec((NP, H), lambda b, k: (b, 0))
        out_shape = jax.ShapeDtypeStruct((BN, H), jnp.float32)
    return pl.pallas_call(
        functools.partial(_gat_body, apply_ln=apply_ln, final=final),
        grid=grid,
        in_specs=in_specs,
        out_specs=out_specs,
        out_shape=out_shape,
        scratch_shapes=[
            pltpu.VMEM((NP, HEADS * H), jnp.float32),
            pltpu.VMEM((NP, HEADS * 128), jnp.float32),
        ],
        compiler_params=pltpu.CompilerParams(
            dimension_semantics=("arbitrary", "arbitrary")),
    )(c, xw, east, ebst, alst, ead, ebd, nad, bias, lng, lnb)


def _pack_att(att):
    m = jnp.zeros((HEADS * H, 128), jnp.float32)
    for h in range(HEADS):
        m = m.at[h * H:(h + 1) * H, h].set(att[h])
    return m


def _layer(x, c, w, att_src, att_dst, bias, lng, lnb, apply_ln,
           pad_input=False, final=False):
    asd_s = _pack_att(att_src)
    asd_d = _pack_att(att_dst)
    xw, east, ebst, alst, ead, ebd, nad = _run_proj(x, w, asd_s, asd_d,
                                                    pad_input)
    b2 = jnp.broadcast_to(bias[None, :], (8, H))
    g2 = jnp.broadcast_to(lng[None, :], (8, H))
    l2 = jnp.broadcast_to(lnb[None, :], (8, H))
    return _run_gat(c, xw, east, ebst, alst, ead, ebd, nad, b2, g2, l2,
                    apply_ln, final)


def kernel(node_embeddings, src_index, tgt_index,
           W1, att_src1, att_dst1, bias1,
           W2, att_src2, att_dst2, bias2,
           W3, att_src3, att_dst3, bias3,
           ln1_g, ln1_b, ln2_g, ln2_b):
    src_flat = src_index.reshape(-1).astype(jnp.int32)
    tgt_flat = tgt_index.reshape(-1).astype(jnp.int32)
    c = _build_c(src_flat, tgt_flat)
    x = _layer(node_embeddings, c, W1, att_src1, att_dst1, bias1,
               ln1_g, ln1_b, True, pad_input=True)
    x = _layer(x, c, W2, att_src2, att_dst2, bias2, ln2_g, ln2_b, True)
    x = _layer(x, c, W3, att_src3, att_dst3, bias3, ln1_g, ln1_b, False,
               final=True)
    return x


# speedup vs baseline: 1.2479x; 1.0492x over previous
import functools

import jax
import jax.numpy as jnp
from jax import lax
from jax.experimental import pallas as pl
from jax.experimental.pallas import tpu as pltpu
from jax.experimental.pallas import tpu_sc as plsc

B = 4
N = 2500
NP = 2560
BN = B * NP
H = 256
HEADS = 4
E = 40000
KT = 512
MT = 1280
NMT = NP // MT
NKT = NP // KT

ROWS_PER_TILE = 40
NTILES = 32
HALF = ROWS_PER_TILE * NTILES
ECHUNK = 4000
BUFW = ROWS_PER_TILE * NP


def _sc_build_c_body(src_hbm, tgt_hbm, c_hbm, buf, sstage0, sstage1,
                     tstage0, tstage1, dsem):
    sstage = (sstage0, sstage1)
    tstage = (tstage0, tstage1)
    cid = lax.axis_index("c")
    sid = lax.axis_index("s")
    wid = sid * 2 + cid
    ones = jnp.full((16,), 1.0, jnp.float32)
    zeros = jnp.zeros((16,), jnp.float32)
    lane = lax.iota(jnp.int32, 16)
    nchunks = E // ECHUNK

    def start_chunk(b, ch, par):
        eoff = b * E + ch * ECHUNK
        cs = pltpu.make_async_copy(
            src_hbm.at[pl.ds(eoff, ECHUNK)], sstage[par], dsem)
        cs.start()
        ct = pltpu.make_async_copy(
            tgt_hbm.at[pl.ds(eoff, ECHUNK)], tstage[par], dsem)
        ct.start()
        return cs, ct

    for p in range(2 * B):
        b = p // 2
        half = p % 2
        lo = half * HALF + wid * ROWS_PER_TILE

        pend = start_chunk(b, 0, 0)

        def zbody(i, _):
            for z in range(8):
                buf[pl.ds(i * 128 + z * 16, 16)] = zeros
            return 0
        lax.fori_loop(0, BUFW // 128, zbody, 0)

        for g in range(3):
            j = g * 16 + lane
            didx = j * NP + lo + j
            dmask = j < ROWS_PER_TILE
            didx = jnp.where(dmask, didx, 0)
            plsc.addupdate_scatter(buf, [didx], ones, mask=dmask)

        for ch in range(nchunks):
            par = ch % 2
            pend[0].wait()
            pend[1].wait()
            if ch + 1 < nchunks:
                pend = start_chunk(b, ch + 1, 1 - par)

            def ebody(i, _):
                for u in range(4):
                    o = i * 64 + u * 16
                    tg = tstage[par][pl.ds(o, 16)]
                    sr = sstage[par][pl.ds(o, 16)]
                    ridx = tg - lo
                    m = ridx.astype(jnp.uint32) < ROWS_PER_TILE
                    fidx = jnp.where(m, ridx * NP + sr, 0)
                    plsc.addupdate_scatter(buf, [fidx], ones, mask=m)
                return 0
            lax.fori_loop(0, ECHUNK // 64, ebody, 0)

        row0 = b * NP + half * HALF + wid * ROWS_PER_TILE
        pltpu.sync_copy(buf, c_hbm.at[pl.ds(row0 * NP, BUFW)])


def _build_c(src_flat, tgt_flat):
    mesh = plsc.VectorSubcoreMesh(core_axis_name="c", subcore_axis_name="s")
    kern = functools.partial(
        pl.kernel,
        mesh=mesh,
        out_type=jax.ShapeDtypeStruct((BN * NP,), jnp.float32),
        scratch_types=[
            pltpu.VMEM((BUFW,), jnp.float32),
            pltpu.VMEM((ECHUNK,), jnp.int32),
            pltpu.VMEM((ECHUNK,), jnp.int32),
            pltpu.VMEM((ECHUNK,), jnp.int32),
            pltpu.VMEM((ECHUNK,), jnp.int32),
            pltpu.SemaphoreType.DMA,
        ],
        compiler_params=pltpu.CompilerParams(needs_layout_passes=False),
    )(_sc_build_c_body)
    return kern(src_flat, tgt_flat).reshape(BN, NP)


def _proj_body(x_ref, w_ref, asd_s_ref, asd_d_ref,
               xw_ref, east_ref, ebst_ref, alst_ref,
               ead_ref, ebd_ref, nad_ref, xs_ref, *, pad_input):
    if pad_input:
        xs_ref[pl.ds(0, N), :] = x_ref[0]
        xs_ref[pl.ds(N, NP - N), :] = jnp.zeros((NP - N, H), jnp.float32)
        x = xs_ref[...]
    else:
        x = x_ref[...]
    xw = jnp.dot(x, w_ref[...], preferred_element_type=jnp.float32)
    xw_ref[...] = xw.astype(jnp.bfloat16)
    als = jnp.dot(xw, asd_s_ref[...], preferred_element_type=jnp.float32)
    ald = jnp.dot(xw, asd_d_ref[...], preferred_element_type=jnp.float32)
    ms = jnp.max(als, axis=0, keepdims=True)
    t = ms + ald
    mp = jnp.where(t > 0, t, 0.2 * t)
    ead_ref[...] = jnp.exp(ald - mp).astype(jnp.bfloat16)
    ebd_ref[...] = jnp.exp(0.2 * ald - mp).astype(jnp.bfloat16)
    nad_ref[...] = (-ald).astype(jnp.bfloat16)
    alt = jnp.transpose(als)
    east_ref[...] = jnp.exp(alt[0:8, :]).astype(jnp.bfloat16)
    ebst_ref[...] = jnp.exp(0.2 * alt[0:8, :]).astype(jnp.bfloat16)
    alst_ref[...] = alt[0:8, :].astype(jnp.bfloat16)


def _run_proj(x, w, asd_s, asd_d, pad_input):
    f32 = jnp.float32
    bf16 = jnp.bfloat16
    out_shapes = (
        jax.ShapeDtypeStruct((BN, HEADS * H), bf16),
        jax.ShapeDtypeStruct((B * 8, NP), bf16),
        jax.ShapeDtypeStruct((B * 8, NP), bf16),
        jax.ShapeDtypeStruct((B * 8, NP), bf16),
        jax.ShapeDtypeStruct((BN, 128), bf16),
        jax.ShapeDtypeStruct((BN, 128), bf16),
        jax.ShapeDtypeStruct((BN, 128), bf16),
    )
    grid = (B,)
    if pad_input:
        xspec = pl.BlockSpec((1, N, H), lambda b: (b, 0, 0))
    else:
        xspec = pl.BlockSpec((NP, H), lambda b: (b, 0))
    in_specs = [
        xspec,
        pl.BlockSpec((H, HEADS * H), lambda b: (0, 0)),
        pl.BlockSpec((HEADS * H, 128), lambda b: (0, 0)),
        pl.BlockSpec((HEADS * H, 128), lambda b: (0, 0)),
    ]
    out_specs = (
        pl.BlockSpec((NP, HEADS * H), lambda b: (b, 0)),
        pl.BlockSpec((8, NP), lambda b: (b, 0)),
        pl.BlockSpec((8, NP), lambda b: (b, 0)),
        pl.BlockSpec((8, NP), lambda b: (b, 0)),
        pl.BlockSpec((NP, 128), lambda b: (b, 0)),
        pl.BlockSpec((NP, 128), lambda b: (b, 0)),
        pl.BlockSpec((NP, 128), lambda b: (b, 0)),
    )
    return pl.pallas_call(
        functools.partial(_proj_body, pad_input=pad_input),
        grid=grid,
        in_specs=in_specs,
        out_specs=out_specs,
        out_shape=out_shapes,
        scratch_shapes=[pltpu.VMEM((NP, H), f32)],
        compiler_params=pltpu.CompilerParams(
            dimension_semantics=("arbitrary",)),
    )(x, w, asd_s, asd_d)


def _gat_body(c_ref, xw_ref, east_ref, ebst_ref, alst_ref,
              ead_ref, ebd_ref, nad_ref, bias_ref, lng_ref, lnb_ref,
              out_ref, acc_ref, ssum_ref, *, apply_ln, final):
    k = pl.program_id(1)

    @pl.when(k == 0)
    def _init():
        acc_ref[...] = jnp.zeros_like(acc_ref)
        ssum_ref[...] = jnp.zeros_like(ssum_ref)

    c = c_ref[...].astype(jnp.bfloat16)
    xw = xw_ref[...]
    east = east_ref[...]
    ebst = ebst_ref[...]
    alst = alst_ref[...]
    ead = ead_ref[...]
    ebd = ebd_ref[...]
    nad = nad_ref[...]

    for h in range(HEADS):
        eas = east[h:h + 1, :]
        ebs = ebst[h:h + 1, :]
        als = alst[h:h + 1, :]
        eadh = ead[:, h:h + 1]
        ebdh = ebd[:, h:h + 1]
        nadh = nad[:, h:h + 1]
        cond = als > nadh
        q = jnp.where(cond, eas * eadh, ebs * ebdh)
        a = c * q
        acc_ref[:, h * H:(h + 1) * H] += jnp.dot(
            a, xw[:, h * H:(h + 1) * H], preferred_element_type=jnp.float32)
        part = (a[:, 0:128] + a[:, 128:256]) + (a[:, 256:384] + a[:, 384:512])
        ssum_ref[:, h * 128:(h + 1) * 128] += part.astype(jnp.float32)

    @pl.when(k == NKT - 1)
    def _finalize():
        o = jnp.zeros((NP, H), jnp.float32)
        for h in range(HEADS):
            rs = jnp.sum(ssum_ref[:, h * 128:(h + 1) * 128], axis=1,
                         keepdims=True)
            o = o + acc_ref[:, h * H:(h + 1) * H] / (rs + 1e-16)
        o = o * (1.0 / HEADS) + bias_ref[0:1, :]
        if apply_ln:
            mu = jnp.mean(o, axis=1, keepdims=True)
            d = o - mu
            var = jnp.mean(d * d, axis=1, keepdims=True)
            o = d * jax.lax.rsqrt(var + 1e-5) * lng_ref[0:1, :] + lnb_ref[0:1, :]
            o = jnp.maximum(o, 0.0)
        if final:
            out_ref[0] = o[0:N, :]
        else:
            out_ref[...] = o


def _run_gat(c, xw, east, ebst, alst, ead, ebd, nad, bias, lng, lnb,
             apply_ln, final):
    grid = (B, NKT)
    in_specs = [
        pl.BlockSpec((NP, KT), lambda b, k: (b, k)),
        pl.BlockSpec((KT, HEADS * H), lambda b, k: (b * NKT + k, 0)),
        pl.BlockSpec((8, KT), lambda b, k: (b, k)),
        pl.BlockSpec((8, KT), lambda b, k: (b, k)),
        pl.BlockSpec((8, KT), lambda b, k: (b, k)),
        pl.BlockSpec((NP, 128), lambda b, k: (b, 0)),
        pl.BlockSpec((NP, 128), lambda b, k: (b, 0)),
        pl.BlockSpec((NP, 128), lambda b, k: (b, 0)),
        pl.BlockSpec((8, H), lambda b, k: (0, 0)),
        pl.BlockSpec((8, H), lambda b, k: (0, 0)),
        pl.BlockSpec((8, H), lambda b, k: (0, 0)),
    ]
    if final:
        out_specs = pl.BlockSpec((1, N, H), lambda b, k: (b, 0, 0))
        out_shape = jax.ShapeDtypeStruct((B, N, H), jnp.float32)
    else:
        out_specs = pl.BlockSpec((NP, H), lambda b, k: (b, 0))
        out_shape = jax.ShapeDtypeStruct((BN, H), jnp.float32)
    return pl.pallas_call(
        functools.partial(_gat_body, apply_ln=apply_ln, final=final),
        grid=grid,
        in_specs=in_specs,
        out_specs=out_specs,
        out_shape=out_shape,
        scratch_shapes=[
            pltpu.VMEM((NP, HEADS * H), jnp.float32),
            pltpu.VMEM((NP, HEADS * 128), jnp.float32),
        ],
        compiler_params=pltpu.CompilerParams(
            dimension_semantics=("arbitrary", "arbitrary")),
    )(c, xw, east, ebst, alst, ead, ebd, nad, bias, lng, lnb)


def _pack_att(att):
    m = jnp.zeros((HEADS * H, 128), jnp.float32)
    for h in range(HEADS):
        m = m.at[h * H:(h + 1) * H, h].set(att[h])
    return m


def _layer(x, c, w, att_src, att_dst, bias, lng, lnb, apply_ln,
           pad_input=False, final=False):
    asd_s = _pack_att(att_src)
    asd_d = _pack_att(att_dst)
    xw, east, ebst, alst, ead, ebd, nad = _run_proj(x, w, asd_s, asd_d,
                                                    pad_input)
    b2 = jnp.broadcast_to(bias[None, :], (8, H))
    g2 = jnp.broadcast_to(lng[None, :], (8, H))
    l2 = jnp.broadcast_to(lnb[None, :], (8, H))
    return _run_gat(c, xw, east, ebst, alst, ead, ebd, nad, b2, g2, l2,
                    apply_ln, final)


def kernel(node_embeddings, src_index, tgt_index,
           W1, att_src1, att_dst1, bias1,
           W2, att_src2, att_dst2, bias2,
           W3, att_src3, att_dst3, bias3,
           ln1_g, ln1_b, ln2_g, ln2_b):
    src_flat = src_index.reshape(-1).astype(jnp.int32)
    tgt_flat = tgt_index.reshape(-1).astype(jnp.int32)
    c = _build_c(src_flat, tgt_flat)
    x = _layer(node_embeddings, c, W1, att_src1, att_dst1, bias1,
               ln1_g, ln1_b, True, pad_input=True)
    x = _layer(x, c, W2, att_src2, att_dst2, bias2, ln2_g, ln2_b, True)
    x = _layer(x, c, W3, att_src3, att_dst3, bias3, ln1_g, ln1_b, False,
               final=True)
    return x
